# baseline (device time: 100386 ns/iter reference)
import jax
import jax.numpy as jnp
from jax import lax
from jax.experimental import pallas as pl
from jax.experimental.pallas import tpu as pltpu

N_DEV = 4
CAPACITY = 204


def kernel(x, router_W, route_idx, expert_W):
    del router_W
    n_tok, d = x.shape
    e_per, _, h_dim = expert_W.shape
    n_exp = N_DEV * e_per
    n_hops = N_DEV - 1

    def body(x_ref, route_ref, w_ref, out_ref,
             w_comm, r_comm, w_send, w_recv, r_send, r_recv):
        my = lax.axis_index("i")
        left = lax.rem(my + N_DEV - 1, N_DEV)
        right = lax.rem(my + 1, N_DEV)

        barrier = pltpu.get_barrier_semaphore()
        for nbr in (left, right):
            pl.semaphore_signal(
                barrier, inc=1,
                device_id=(nbr,), device_id_type=pl.DeviceIdType.MESH,
            )
        pl.semaphore_wait(barrier, 2)

        def start_hop(h):
            rw = pltpu.make_async_remote_copy(
                src_ref=w_ref if h == 0 else w_comm.at[h - 1],
                dst_ref=w_comm.at[h],
                send_sem=w_send.at[h], recv_sem=w_recv.at[h],
                device_id=(right,), device_id_type=pl.DeviceIdType.MESH,
            )
            rr = pltpu.make_async_remote_copy(
                src_ref=route_ref if h == 0 else r_comm.at[h - 1],
                dst_ref=r_comm.at[h],
                send_sem=r_send.at[h], recv_sem=r_recv.at[h],
                device_id=(right,), device_id_type=pl.DeviceIdType.MESH,
            )
            rw.start()
            rr.start()
            return rw, rr

        route = route_ref[...]
        x_val = x_ref[...]

        def compute_block(w_get, s):
            acc = out_ref[...]
            for el in range(e_per):
                e = s * e_per + el
                m = (route == e).astype(jnp.float32)
                acc = acc + jnp.dot(
                    x_val * m, w_get(el),
                    preferred_element_type=jnp.float32,
                )
            out_ref[...] = acc

        out_ref[...] = jnp.zeros_like(out_ref)

        rw, rr = start_hop(0)
        compute_block(lambda el: w_ref[el], my)

        for hop in range(1, N_DEV):
            rw.wait()
            rr.wait()
            if hop < n_hops:
                rw, rr = start_hop(hop)
            s = lax.rem(my - hop + N_DEV, N_DEV)
            compute_block(lambda el, _h=hop: w_comm[_h - 1, el], s)

        iota_e = lax.broadcasted_iota(jnp.int32, (n_tok, n_exp), 1)
        onehot = (route == iota_e).astype(jnp.float32)
        row = lax.broadcasted_iota(jnp.int32, (n_tok, n_tok), 0)
        col = lax.broadcasted_iota(jnp.int32, (n_tok, n_tok), 1)
        tril = (col < row).astype(jnp.float32)
        rank = jnp.dot(tril, onehot, preferred_element_type=jnp.float32)
        off = jnp.zeros((1, n_exp), jnp.float32)
        for hop in range(n_hops):
            s = lax.rem(my - hop - 1 + N_DEV, N_DEV)
            oh = (r_comm[hop] == iota_e).astype(jnp.float32)
            hist = jnp.sum(oh, axis=0, keepdims=True)
            off = off + jnp.where(s < my, 1.0, 0.0) * hist
        g = rank + off
        g_tok = jnp.sum(onehot * g, axis=1, keepdims=True)
        survive = (g_tok < float(CAPACITY)).astype(jnp.float32)
        out_ref[...] = out_ref[...] * survive

    return pl.pallas_call(
        body,
        out_shape=jax.ShapeDtypeStruct((n_tok, h_dim), jnp.float32),
        in_specs=[
            pl.BlockSpec(memory_space=pltpu.VMEM),
            pl.BlockSpec(memory_space=pltpu.VMEM),
            pl.BlockSpec(memory_space=pltpu.VMEM),
        ],
        out_specs=pl.BlockSpec(memory_space=pltpu.VMEM),
        scratch_shapes=[
            pltpu.VMEM((N_DEV - 1, e_per, d, h_dim), jnp.float32),
            pltpu.VMEM((N_DEV - 1, n_tok, 1), jnp.int32),
            pltpu.SemaphoreType.DMA((N_DEV - 1,)),
            pltpu.SemaphoreType.DMA((N_DEV - 1,)),
            pltpu.SemaphoreType.DMA((N_DEV - 1,)),
            pltpu.SemaphoreType.DMA((N_DEV - 1,)),
        ],
        compiler_params=pltpu.CompilerParams(collective_id=0),
    )(x, route_idx, expert_W)
